# baseline (speedup 1.0000x reference)
import jax
import jax.numpy as jnp
from jax.experimental import pallas as pl
from jax.experimental.pallas import tpu as pltpu

_ROWS = 8192
_COLS = 4096
_BLOCK_ROWS = 256


def _masked_mse_kernel(yp_ref, yt_ref, m_ref, sum_ref, cnt_ref):
    i = pl.program_id(0)

    @pl.when(i == 0)
    def _init():
        sum_ref[0, 0] = jnp.float32(0.0)
        cnt_ref[0, 0] = jnp.float32(0.0)

    d = yp_ref[...] - yt_ref[...]
    m = m_ref[...]
    sq = jnp.where(m, d * d, jnp.float32(0.0))
    sum_ref[0, 0] += jnp.sum(sq)
    cnt_ref[0, 0] += jnp.sum(m.astype(jnp.float32))


def kernel(y_pred, y_true, mask):
    yp = y_pred.reshape(_ROWS, _COLS)
    yt = y_true.reshape(_ROWS, _COLS)
    m = mask.reshape(_ROWS, _COLS)

    grid = (_ROWS // _BLOCK_ROWS,)
    in_spec = pl.BlockSpec((_BLOCK_ROWS, _COLS), lambda i: (i, 0))
    out_spec = pl.BlockSpec(memory_space=pltpu.SMEM)

    s, n = pl.pallas_call(
        _masked_mse_kernel,
        grid=grid,
        in_specs=[in_spec, in_spec, in_spec],
        out_specs=[out_spec, out_spec],
        out_shape=[
            jax.ShapeDtypeStruct((1, 1), jnp.float32),
            jax.ShapeDtypeStruct((1, 1), jnp.float32),
        ],
    )(yp, yt, m)
    return s[0, 0] / n[0, 0]
